# baseline (speedup 1.0000x reference)
import functools

import jax
import jax.numpy as jnp
from jax import lax
from jax.experimental import pallas as pl
from jax.experimental.pallas import tpu as pltpu
from jax.experimental.pallas import tpu_sc as plsc

N, B, H = 100000, 256, 512
R = 5000
NUM_BLOCKS = N // R

NC, NS = 2, 16
NW = NC * NS
CH = 32
NROW = N // CH
RPW = -(-NROW // NW)
NROW_PAD = RPW * NW
HB = 512


def _sc_counts_body(idx3, out_n, idx_v, acc_v):
    c = lax.axis_index("c")
    s = lax.axis_index("s")
    w = s * NC + c

    pltpu.sync_copy(idx3.at[w], idx_v)

    zeros16 = jnp.zeros((16,), jnp.float32)
    for k in range(HB // 16):
        acc_v[pl.ds(k * 16, 16)] = zeros16

    ones16 = jnp.ones((16,), jnp.float32)
    for j in range(RPW):
        plsc.addupdate_scatter(acc_v, [idx_v[j, pl.ds(0, 16)]], ones16)
        plsc.addupdate_scatter(acc_v, [idx_v[j, pl.ds(16, 16)]], ones16)

    pltpu.sync_copy(acc_v, out_n.at[w])


_sc_counts = functools.partial(
    pl.kernel,
    out_type=jax.ShapeDtypeStruct((NW, HB), jnp.float32),
    mesh=plsc.VectorSubcoreMesh(core_axis_name="c", subcore_axis_name="s"),
    scratch_types=[
        pltpu.VMEM((RPW, CH), jnp.int32),
        pltpu.VMEM((HB,), jnp.float32),
    ],
    compiler_params=pltpu.CompilerParams(needs_layout_passes=False),
)


def _dense_body(ids_ref, nm_ref, nc_ref, om_ref, oc_ref):
    i = pl.program_id(0)
    ids = ids_ref[0, 0, :]
    seg = jax.lax.broadcasted_iota(jnp.int32, (B, R), 0)
    onehot = (seg == ids[None, :]).astype(jnp.float32)

    pm = jnp.dot(onehot, nm_ref[...], preferred_element_type=jnp.float32)
    pc = jnp.dot(onehot, nc_ref[...], preferred_element_type=jnp.float32)

    @pl.when(i == 0)
    def _set():
        om_ref[...] = pm
        oc_ref[...] = pc

    @pl.when(i != 0)
    def _add():
        om_ref[...] += pm
        oc_ref[...] += pc


def _finale_body(sm_ref, sc_ref, cn_ref, pm_ref, pc_ref,
                 o_tot_ref, o_m_ref, o_c_ref):
    eps = 1e-8
    ones_w = jnp.ones((NW, 1), jnp.float32)
    cnt_col = jax.lax.dot_general(
        cn_ref[...], ones_w, (((0,), (0,)), ((), ())),
        preferred_element_type=jnp.float32)[0:B, :]
    inv = 1.0 / jnp.maximum(cnt_col, 1.0)
    mean_m = sm_ref[...] * inv
    mean_c = sc_ref[...] * inv

    def cos_dist_mean(a, p):
        num = jnp.sum(a * p, axis=1, keepdims=True)
        na = jnp.maximum(jnp.sqrt(jnp.sum(a * a, axis=1, keepdims=True)), eps)
        nb = jnp.maximum(jnp.sqrt(jnp.sum(p * p, axis=1, keepdims=True)), eps)
        cos = num / (na * nb)
        return jnp.mean(1.0 - cos)

    lm = cos_dist_mean(mean_m, pm_ref[...])
    lc = cos_dist_mean(mean_c, pc_ref[...])
    o_m_ref[...] = jnp.reshape(lm, (1, 1))
    o_c_ref[...] = jnp.reshape(lc, (1, 1))
    o_tot_ref[...] = jnp.reshape((lm + lc) * 0.5, (1, 1))


def kernel(node_mrna, node_cnv, pooled_mrna, pooled_cnv, gene_batch):
    idx3 = jnp.pad(gene_batch, (0, NROW_PAD * CH - N),
                   constant_values=B).reshape(NW, RPW, CH)
    cnt = _sc_counts(_sc_counts_body)(idx3)

    ids3 = gene_batch.reshape(NUM_BLOCKS, 1, R)
    sums = jax.ShapeDtypeStruct((B, H), jnp.float32)
    sum_m, sum_c = pl.pallas_call(
        _dense_body,
        grid=(NUM_BLOCKS,),
        in_specs=[
            pl.BlockSpec((1, 1, R), lambda i: (i, 0, 0)),
            pl.BlockSpec((R, H), lambda i: (i, 0)),
            pl.BlockSpec((R, H), lambda i: (i, 0)),
        ],
        out_specs=[
            pl.BlockSpec((B, H), lambda i: (0, 0)),
            pl.BlockSpec((B, H), lambda i: (0, 0)),
        ],
        out_shape=[sums, sums],
    )(ids3, node_mrna, node_cnv)

    scalar = jax.ShapeDtypeStruct((1, 1), jnp.float32)
    tot, lm, lc = pl.pallas_call(
        _finale_body,
        out_shape=[scalar, scalar, scalar],
    )(sum_m, sum_c, cnt, pooled_mrna, pooled_cnv)
    return (tot[0, 0], lm[0, 0], lc[0, 0])


# speedup vs baseline: 1.0101x; 1.0101x over previous
import functools

import jax
import jax.numpy as jnp
from jax import lax
from jax.experimental import pallas as pl
from jax.experimental.pallas import tpu as pltpu
from jax.experimental.pallas import tpu_sc as plsc

N, B, H = 100000, 256, 512
R = 4000
NUM_BLOCKS = N // R

NC, NS = 2, 16
NW = NC * NS
CH = 32
NROW = N // CH
RPW = -(-NROW // NW)
NROW_PAD = RPW * NW
HB = 512


def _sc_counts_body(idx3, out_n, idx_v, acc_v):
    c = lax.axis_index("c")
    s = lax.axis_index("s")
    w = s * NC + c

    pltpu.sync_copy(idx3.at[w], idx_v)

    zeros16 = jnp.zeros((16,), jnp.float32)
    for k in range(HB // 16):
        acc_v[pl.ds(k * 16, 16)] = zeros16

    ones16 = jnp.ones((16,), jnp.float32)
    for j in range(RPW):
        plsc.addupdate_scatter(acc_v, [idx_v[j, pl.ds(0, 16)]], ones16)
        plsc.addupdate_scatter(acc_v, [idx_v[j, pl.ds(16, 16)]], ones16)

    pltpu.sync_copy(acc_v, out_n.at[w])


_sc_counts = functools.partial(
    pl.kernel,
    out_type=jax.ShapeDtypeStruct((NW, HB), jnp.float32),
    mesh=plsc.VectorSubcoreMesh(core_axis_name="c", subcore_axis_name="s"),
    scratch_types=[
        pltpu.VMEM((RPW, CH), jnp.int32),
        pltpu.VMEM((HB,), jnp.float32),
    ],
    compiler_params=pltpu.CompilerParams(needs_layout_passes=False),
)


def _dense_body(ids_ref, nm_ref, nc_ref, om_ref, oc_ref):
    i = pl.program_id(0)
    ids = ids_ref[0, 0, :]
    seg = jax.lax.broadcasted_iota(jnp.int32, (B, R), 0)
    onehot = (seg == ids[None, :]).astype(jnp.float32)

    pm = jnp.dot(onehot, nm_ref[...], preferred_element_type=jnp.float32)
    pc = jnp.dot(onehot, nc_ref[...], preferred_element_type=jnp.float32)

    @pl.when(i == 0)
    def _set():
        om_ref[...] = pm
        oc_ref[...] = pc

    @pl.when(i != 0)
    def _add():
        om_ref[...] += pm
        oc_ref[...] += pc


def _finale_body(sm_ref, sc_ref, cn_ref, pm_ref, pc_ref,
                 o_tot_ref, o_m_ref, o_c_ref):
    eps = 1e-8
    ones_w = jnp.ones((NW, 1), jnp.float32)
    cnt_col = jax.lax.dot_general(
        cn_ref[...], ones_w, (((0,), (0,)), ((), ())),
        preferred_element_type=jnp.float32)[0:B, :]
    inv = 1.0 / jnp.maximum(cnt_col, 1.0)
    mean_m = sm_ref[...] * inv
    mean_c = sc_ref[...] * inv

    def cos_dist_mean(a, p):
        num = jnp.sum(a * p, axis=1, keepdims=True)
        na = jnp.maximum(jnp.sqrt(jnp.sum(a * a, axis=1, keepdims=True)), eps)
        nb = jnp.maximum(jnp.sqrt(jnp.sum(p * p, axis=1, keepdims=True)), eps)
        cos = num / (na * nb)
        return jnp.mean(1.0 - cos)

    lm = cos_dist_mean(mean_m, pm_ref[...])
    lc = cos_dist_mean(mean_c, pc_ref[...])
    o_m_ref[...] = jnp.reshape(lm, (1, 1))
    o_c_ref[...] = jnp.reshape(lc, (1, 1))
    o_tot_ref[...] = jnp.reshape((lm + lc) * 0.5, (1, 1))


def kernel(node_mrna, node_cnv, pooled_mrna, pooled_cnv, gene_batch):
    idx3 = jnp.pad(gene_batch, (0, NROW_PAD * CH - N),
                   constant_values=B).reshape(NW, RPW, CH)
    cnt = _sc_counts(_sc_counts_body)(idx3)

    ids3 = gene_batch.reshape(NUM_BLOCKS, 1, R)
    sums = jax.ShapeDtypeStruct((B, H), jnp.float32)
    sum_m, sum_c = pl.pallas_call(
        _dense_body,
        grid=(NUM_BLOCKS,),
        in_specs=[
            pl.BlockSpec((1, 1, R), lambda i: (i, 0, 0)),
            pl.BlockSpec((R, H), lambda i: (i, 0)),
            pl.BlockSpec((R, H), lambda i: (i, 0)),
        ],
        out_specs=[
            pl.BlockSpec((B, H), lambda i: (0, 0)),
            pl.BlockSpec((B, H), lambda i: (0, 0)),
        ],
        out_shape=[sums, sums],
    )(ids3, node_mrna, node_cnv)

    scalar = jax.ShapeDtypeStruct((1, 1), jnp.float32)
    tot, lm, lc = pl.pallas_call(
        _finale_body,
        out_shape=[scalar, scalar, scalar],
    )(sum_m, sum_c, cnt, pooled_mrna, pooled_cnv)
    return (tot[0, 0], lm[0, 0], lc[0, 0])


# speedup vs baseline: 1.0276x; 1.0174x over previous
import functools

import jax
import jax.numpy as jnp
from jax import lax
from jax.experimental import pallas as pl
from jax.experimental.pallas import tpu as pltpu
from jax.experimental.pallas import tpu_sc as plsc

N, B, H = 100000, 256, 512
R = 4000
NUM_BLOCKS = N // R

NC, NS = 2, 16
NW = NC * NS
CH = 32
NROW = N // CH
RPW = -(-NROW // NW)
NROW_PAD = RPW * NW
HB = 512


def _sc_counts_body(ids_hbm, out_n, idx_v, acc_v):
    c = lax.axis_index("c")
    s = lax.axis_index("s")
    w = s * NC + c
    is_last = w == NW - 1
    base = jnp.where(is_last, N - RPW * CH, w * RPW * CH)
    j0 = jnp.where(is_last, NROW_PAD - NROW, 0)

    pltpu.sync_copy(ids_hbm.at[pl.ds(base, RPW * CH)], idx_v)

    zeros16 = jnp.zeros((16,), jnp.float32)
    for k in range(HB // 16):
        acc_v[pl.ds(k * 16, 16)] = zeros16

    ones16 = jnp.ones((16,), jnp.float32)
    for j in range(RPW):
        m = jax.lax.broadcast(j >= j0, (16,))
        plsc.addupdate_scatter(acc_v, [idx_v[pl.ds(j * CH, 16)]], ones16,
                               mask=m)
        plsc.addupdate_scatter(acc_v, [idx_v[pl.ds(j * CH + 16, 16)]], ones16,
                               mask=m)

    pltpu.sync_copy(acc_v, out_n.at[w])


_sc_counts = functools.partial(
    pl.kernel,
    out_type=jax.ShapeDtypeStruct((NW, HB), jnp.float32),
    mesh=plsc.VectorSubcoreMesh(core_axis_name="c", subcore_axis_name="s"),
    scratch_types=[
        pltpu.VMEM((RPW * CH,), jnp.int32),
        pltpu.VMEM((HB,), jnp.float32),
    ],
    compiler_params=pltpu.CompilerParams(needs_layout_passes=False),
)


def _dense_body(ids_ref, nm_ref, nc_ref, om_ref, oc_ref):
    i = pl.program_id(0)
    ids = ids_ref[0, 0, :]
    seg = jax.lax.broadcasted_iota(jnp.int32, (B, R), 0)
    onehot = (seg == ids[None, :]).astype(jnp.float32)

    pm = jnp.dot(onehot, nm_ref[...], preferred_element_type=jnp.float32)
    pc = jnp.dot(onehot, nc_ref[...], preferred_element_type=jnp.float32)

    @pl.when(i == 0)
    def _set():
        om_ref[...] = pm
        oc_ref[...] = pc

    @pl.when(i != 0)
    def _add():
        om_ref[...] += pm
        oc_ref[...] += pc


def _finale_body(sm_ref, sc_ref, cn_ref, pm_ref, pc_ref,
                 o_tot_ref, o_m_ref, o_c_ref):
    eps = 1e-8
    ones_w = jnp.ones((NW, 1), jnp.float32)
    cnt_col = jax.lax.dot_general(
        cn_ref[...], ones_w, (((0,), (0,)), ((), ())),
        preferred_element_type=jnp.float32)[0:B, :]
    inv = 1.0 / jnp.maximum(cnt_col, 1.0)
    mean_m = sm_ref[...] * inv
    mean_c = sc_ref[...] * inv

    def cos_dist_mean(a, p):
        num = jnp.sum(a * p, axis=1, keepdims=True)
        na = jnp.maximum(jnp.sqrt(jnp.sum(a * a, axis=1, keepdims=True)), eps)
        nb = jnp.maximum(jnp.sqrt(jnp.sum(p * p, axis=1, keepdims=True)), eps)
        cos = num / (na * nb)
        return jnp.mean(1.0 - cos)

    lm = cos_dist_mean(mean_m, pm_ref[...])
    lc = cos_dist_mean(mean_c, pc_ref[...])
    o_m_ref[...] = jnp.reshape(lm, (1, 1))
    o_c_ref[...] = jnp.reshape(lc, (1, 1))
    o_tot_ref[...] = jnp.reshape((lm + lc) * 0.5, (1, 1))


def kernel(node_mrna, node_cnv, pooled_mrna, pooled_cnv, gene_batch):
    cnt = _sc_counts(_sc_counts_body)(gene_batch)

    ids3 = gene_batch.reshape(NUM_BLOCKS, 1, R)
    sums = jax.ShapeDtypeStruct((B, H), jnp.float32)
    sum_m, sum_c = pl.pallas_call(
        _dense_body,
        grid=(NUM_BLOCKS,),
        in_specs=[
            pl.BlockSpec((1, 1, R), lambda i: (i, 0, 0)),
            pl.BlockSpec((R, H), lambda i: (i, 0)),
            pl.BlockSpec((R, H), lambda i: (i, 0)),
        ],
        out_specs=[
            pl.BlockSpec((B, H), lambda i: (0, 0)),
            pl.BlockSpec((B, H), lambda i: (0, 0)),
        ],
        out_shape=[sums, sums],
    )(ids3, node_mrna, node_cnv)

    scalar = jax.ShapeDtypeStruct((1, 1), jnp.float32)
    tot, lm, lc = pl.pallas_call(
        _finale_body,
        out_shape=[scalar, scalar, scalar],
    )(sum_m, sum_c, cnt, pooled_mrna, pooled_cnv)
    return (tot[0, 0], lm[0, 0], lc[0, 0])
